# Initial kernel scaffold; baseline (speedup 1.0000x reference)
import jax
import jax.numpy as jnp
from jax.experimental import pallas as pl


def kernel(CA, x):
    raise NotImplementedError("write your pallas kernel here")


# speedup vs baseline: 2.3905x; 2.3905x over previous
import jax
import jax.numpy as jnp
from jax import lax
from jax.experimental import pallas as pl
from jax.experimental.pallas import tpu as pltpu
from jax.experimental.pallas import tpu_sc as plsc

NC, NS, L = 2, 16, 16
NW = NC * NS
B, C, W, H = 8, 96, 224, 224
N = W * H
R = B * C
RPW = R // NW
NV = N // L
NBINS = 256
CH = 6272
NCHUNK = N // CH
MIN_I32 = -2147483648


def _wid():
    return lax.axis_index("s") * NC + lax.axis_index("c")


def _to_key(v16_f32):
    b = lax.bitcast_convert_type(v16_f32, jnp.int32)
    s = jnp.right_shift(b, 31)
    return jnp.bitwise_xor(b, jnp.bitwise_or(s, jnp.int32(MIN_I32)))


def _from_key(k16_i32):
    s = jnp.right_shift(k16_i32, 31)
    m = jnp.bitwise_or(jnp.bitwise_not(s), jnp.int32(MIN_I32))
    return lax.bitcast_convert_type(jnp.bitwise_xor(k16_i32, m), jnp.float32)


def _digit(k16_i32, shift):
    ku = lax.bitcast_convert_type(k16_i32, jnp.uint32)
    return (jnp.right_shift(ku, shift) & jnp.uint32(255)).astype(jnp.int32)


def _sort_body(x_hbm, sv_hbm, ka, kb, hist, fbuf):
    wid = _wid()
    lanes = jnp.arange(L, dtype=jnp.int32)
    ones = jnp.ones((L,), jnp.int32)
    segbase = lanes * NV

    def row(t, _):
        r = wid * RPW + t
        def load_chunk(ci, _):
            pltpu.sync_copy(x_hbm.at[r, pl.ds(ci * CH, CH)], fbuf)
            def conv(j, _):
                ka[pl.ds(ci * CH + j * L, L)] = _to_key(fbuf[pl.ds(j * L, L)])
                return 0
            lax.fori_loop(0, CH // L, conv, 0)
            return 0
        lax.fori_loop(0, NCHUNK, load_chunk, 0)

        for p in range(4):
            src = ka if p % 2 == 0 else kb
            dst = kb if p % 2 == 0 else ka
            shift = jnp.uint32(8 * p)

            def zero(i, _):
                hist[pl.ds(i * L, L)] = jnp.zeros((L,), jnp.int32)
                return 0
            lax.fori_loop(0, NBINS * L // L, zero, 0)

            def count(i, _):
                k = plsc.load_gather(src, [segbase + i])
                idx = jnp.left_shift(_digit(k, shift), 4) + lanes
                plsc.addupdate_scatter(hist, [idx], ones)
                return 0
            lax.fori_loop(0, NV, count, 0)

            def scan(i, carry):
                v = hist[pl.ds(i * L, L)]
                inc = plsc.cumsum(v)
                hist[pl.ds(i * L, L)] = inc - v + carry
                return carry + jnp.sum(v)
            lax.fori_loop(0, NBINS, scan, jnp.int32(0))

            def permute(i, _):
                k = plsc.load_gather(src, [segbase + i])
                idx = jnp.left_shift(_digit(k, shift), 4) + lanes
                pos = plsc.load_gather(hist, [idx])
                plsc.store_scatter(dst, [pos], k)
                plsc.addupdate_scatter(hist, [idx], ones)
                return 0
            lax.fori_loop(0, NV, permute, 0)

        def store_chunk(ci, _):
            def conv(j, _):
                fbuf[pl.ds(j * L, L)] = _from_key(ka[pl.ds(ci * CH + j * L, L)])
                return 0
            lax.fori_loop(0, CH // L, conv, 0)
            pltpu.sync_copy(fbuf, sv_hbm.at[r, pl.ds(ci * CH, CH)])
            return 0
        lax.fori_loop(0, NCHUNK, store_chunk, 0)
        return 0

    lax.fori_loop(0, RPW, row, 0)


def _combine_body(x_hbm, sv_hbm, alpha_hbm, srcmap_hbm, out_hbm,
                  svb, svp, srcidx, av, xc, oc, sem):
    wid = _wid()
    zeros16 = jnp.zeros((L,), jnp.int32)

    def row(t, _):
        r = wid * RPW + t
        pltpu.sync_copy(sv_hbm.at[r], svb)
        pltpu.sync_copy(srcmap_hbm.at[r], srcidx)
        pltpu.sync_copy(alpha_hbm.at[r], av)
        pltpu.async_copy(sv_hbm.at[srcidx.at[pl.ds(0, 1)]], svp, sem).wait()
        alpha = av[...]

        def chunk(ci, _):
            pltpu.sync_copy(x_hbm.at[r, pl.ds(ci * CH, CH)], xc)

            def elem(j, _):
                xv = xc[pl.ds(j * L, L)]
                def step(s, st):
                    lo, cnt = st
                    half = jnp.right_shift(cnt, 1)
                    mid = lo + half
                    v = plsc.load_gather(svb, [mid])
                    c = v < xv
                    lo2 = jnp.where(c, mid + 1, lo)
                    cnt2 = jnp.where(c, cnt - half - 1, half)
                    return (lo2, cnt2)
                lo, _cnt = lax.fori_loop(
                    0, 16, step,
                    (jnp.zeros((L,), jnp.int32), jnp.full((L,), N, jnp.int32)))
                m = plsc.load_gather(svp, [zeros16, lo])
                oc[pl.ds(j * L, L)] = xv + alpha * (m - xv)
                return 0
            lax.fori_loop(0, CH // L, elem, 0)
            pltpu.sync_copy(oc, out_hbm.at[r, pl.ds(ci * CH, CH)])
            return 0
        lax.fori_loop(0, NCHUNK, chunk, 0)
        return 0

    lax.fori_loop(0, RPW, row, 0)


def kernel(CA, x):
    xr = x.reshape(R, N)

    perm = jnp.arange(B - 1, -1, -1)
    pk1, pk2 = jax.random.split(jax.random.key(42))
    perm_b = perm[: B // 2][jax.random.permutation(pk1, B // 2)]
    perm_a = perm[B // 2:][jax.random.permutation(pk2, B // 2)]
    perm = jnp.concatenate([perm_b, perm_a], axis=0)

    alpha = jax.nn.sigmoid(-999999.0 * (CA + CA[perm] - 0.6))
    alpha16 = jnp.broadcast_to(alpha.reshape(R, 1), (R, L)).astype(jnp.float32)
    srcrow = (perm[:, None] * C + jnp.arange(C)[None, :]).reshape(R)
    srcmap8 = jnp.broadcast_to(srcrow.reshape(R, 1), (R, 8)).astype(jnp.int32)

    mesh = plsc.VectorSubcoreMesh(core_axis_name="c", subcore_axis_name="s")

    sv = pl.kernel(
        _sort_body,
        out_type=jax.ShapeDtypeStruct((R, N), jnp.float32),
        mesh=mesh,
        compiler_params=pltpu.CompilerParams(needs_layout_passes=False),
        scratch_types=[
            pltpu.VMEM((N,), jnp.int32),
            pltpu.VMEM((N,), jnp.int32),
            pltpu.VMEM((NBINS * L,), jnp.int32),
            pltpu.VMEM((CH,), jnp.float32),
        ],
    )(xr)

    out = pl.kernel(
        _combine_body,
        out_type=jax.ShapeDtypeStruct((R, N), jnp.float32),
        mesh=mesh,
        compiler_params=pltpu.CompilerParams(needs_layout_passes=False),
        scratch_types=[
            pltpu.VMEM((N,), jnp.float32),
            pltpu.VMEM((1, N), jnp.float32),
            pltpu.VMEM((8,), jnp.int32),
            pltpu.VMEM((L,), jnp.float32),
            pltpu.VMEM((CH,), jnp.float32),
            pltpu.VMEM((CH,), jnp.float32),
            pltpu.SemaphoreType.DMA,
        ],
    )(xr, sv, alpha16, srcmap8)

    return out.reshape(B, C, W, H)


# speedup vs baseline: 6.1414x; 2.5690x over previous
import jax
import jax.numpy as jnp
from jax import lax
from jax.experimental import pallas as pl
from jax.experimental.pallas import tpu as pltpu
from jax.experimental.pallas import tpu_sc as plsc

NC, NS, L = 2, 16, 16
NW = NC * NS
B, C, W, H = 8, 96, 224, 224
N = W * H
R = B * C
RPW = R // NW
NV = N // L
NBINS = 256
CH = 6272
NCHUNK = N // CH
MIN_I32 = -2147483648


def _wid():
    return lax.axis_index("s") * NC + lax.axis_index("c")


def _to_key(v16_f32):
    b = lax.bitcast_convert_type(v16_f32, jnp.int32)
    s = jnp.right_shift(b, 31)
    return jnp.bitwise_xor(b, jnp.bitwise_or(s, jnp.int32(MIN_I32)))


def _from_key(k16_i32):
    s = jnp.right_shift(k16_i32, 31)
    m = jnp.bitwise_or(jnp.bitwise_not(s), jnp.int32(MIN_I32))
    return lax.bitcast_convert_type(jnp.bitwise_xor(k16_i32, m), jnp.float32)


def _digit(k16_i32, shift):
    ku = lax.bitcast_convert_type(k16_i32, jnp.uint32)
    return (jnp.right_shift(ku, shift) & jnp.uint32(255)).astype(jnp.int32)


def _sort_body(x_hbm, sortflag_hbm, sv_hbm, ka, kb, hist, fbuf, flagv):
    wid = _wid()
    lanes = jnp.arange(L, dtype=jnp.int32)
    ones = jnp.ones((L,), jnp.int32)
    segbase = lanes * NV

    def row(t, _):
        r = wid * RPW + t
        pltpu.sync_copy(sortflag_hbm.at[r], flagv)
        need = jnp.sum(flagv[...]) > 0

        def do_sort():
            _sort_one_row(x_hbm, sv_hbm, r, ka, kb, hist, fbuf,
                          lanes, ones, segbase)
        lax.cond(need, do_sort, lambda: None)
        return 0

    lax.fori_loop(0, RPW, row, 0)


def _sort_one_row(x_hbm, sv_hbm, r, ka, kb, hist, fbuf, lanes, ones, segbase):
        def load_chunk(ci, _):
            pltpu.sync_copy(x_hbm.at[r, pl.ds(ci * CH, CH)], fbuf)
            def conv(j, _):
                ka[pl.ds(ci * CH + j * L, L)] = _to_key(fbuf[pl.ds(j * L, L)])
                return 0
            lax.fori_loop(0, CH // L, conv, 0)
            return 0
        lax.fori_loop(0, NCHUNK, load_chunk, 0)

        for p in range(4):
            src = ka if p % 2 == 0 else kb
            dst = kb if p % 2 == 0 else ka
            shift = jnp.uint32(8 * p)

            def zero(i, _):
                hist[pl.ds(i * L, L)] = jnp.zeros((L,), jnp.int32)
                return 0
            lax.fori_loop(0, NBINS * L // L, zero, 0)

            def count(i, _):
                k = plsc.load_gather(src, [segbase + i])
                idx = jnp.left_shift(_digit(k, shift), 4) + lanes
                plsc.addupdate_scatter(hist, [idx], ones)
                return 0
            lax.fori_loop(0, NV, count, 0)

            def scan(i, carry):
                v = hist[pl.ds(i * L, L)]
                inc = plsc.cumsum(v)
                hist[pl.ds(i * L, L)] = inc - v + carry
                return carry + jnp.sum(v)
            lax.fori_loop(0, NBINS, scan, jnp.int32(0))

            def permute(i, _):
                k = plsc.load_gather(src, [segbase + i])
                idx = jnp.left_shift(_digit(k, shift), 4) + lanes
                pos = plsc.load_gather(hist, [idx])
                plsc.store_scatter(dst, [pos], k)
                plsc.addupdate_scatter(hist, [idx], ones)
                return 0
            lax.fori_loop(0, NV, permute, 0)

        def store_chunk(ci, _):
            def conv(j, _):
                fbuf[pl.ds(j * L, L)] = _from_key(ka[pl.ds(ci * CH + j * L, L)])
                return 0
            lax.fori_loop(0, CH // L, conv, 0)
            pltpu.sync_copy(fbuf, sv_hbm.at[r, pl.ds(ci * CH, CH)])
            return 0
        lax.fori_loop(0, NCHUNK, store_chunk, 0)


def _combine_body(x_hbm, sv_hbm, alpha_hbm, srcmap_hbm, out_hbm,
                  svb, svp, srcidx, av, xc, oc, sem):
    wid = _wid()
    zeros16 = jnp.zeros((L,), jnp.int32)

    def row(t, _):
        r = wid * RPW + t
        pltpu.sync_copy(alpha_hbm.at[r], av)
        a16 = av[...]
        active = jnp.sum(jnp.where(a16 != 0.0, jnp.int32(1), jnp.int32(0))) > 0

        def match():
            pltpu.sync_copy(sv_hbm.at[r], svb)
            pltpu.sync_copy(srcmap_hbm.at[r], srcidx)
            pltpu.async_copy(sv_hbm.at[srcidx.at[pl.ds(0, 1)]], svp, sem).wait()
            alpha = av[...]

            def chunk(ci, _):
                pltpu.sync_copy(x_hbm.at[r, pl.ds(ci * CH, CH)], xc)

                def elem(j, _):
                    xv = xc[pl.ds(j * L, L)]
                    def step(s, st):
                        lo, cnt = st
                        half = jnp.right_shift(cnt, 1)
                        mid = lo + half
                        v = plsc.load_gather(svb, [mid])
                        c = v < xv
                        lo2 = jnp.where(c, mid + 1, lo)
                        cnt2 = jnp.where(c, cnt - half - 1, half)
                        return (lo2, cnt2)
                    lo, _cnt = lax.fori_loop(
                        0, 16, step,
                        (jnp.zeros((L,), jnp.int32),
                         jnp.full((L,), N, jnp.int32)))
                    m = plsc.load_gather(svp, [zeros16, lo])
                    oc[pl.ds(j * L, L)] = xv + alpha * (m - xv)
                    return 0
                lax.fori_loop(0, CH // L, elem, 0)
                pltpu.sync_copy(oc, out_hbm.at[r, pl.ds(ci * CH, CH)])
                return 0
            lax.fori_loop(0, NCHUNK, chunk, 0)

        def copy():
            def chunk(ci, _):
                pltpu.sync_copy(x_hbm.at[r, pl.ds(ci * CH, CH)], xc)
                pltpu.sync_copy(xc, out_hbm.at[r, pl.ds(ci * CH, CH)])
                return 0
            lax.fori_loop(0, NCHUNK, chunk, 0)

        lax.cond(active, match, copy)
        return 0

    lax.fori_loop(0, RPW, row, 0)


def kernel(CA, x):
    xr = x.reshape(R, N)

    perm = jnp.arange(B - 1, -1, -1)
    pk1, pk2 = jax.random.split(jax.random.key(42))
    perm_b = perm[: B // 2][jax.random.permutation(pk1, B // 2)]
    perm_a = perm[B // 2:][jax.random.permutation(pk2, B // 2)]
    perm = jnp.concatenate([perm_b, perm_a], axis=0)

    alpha = jax.nn.sigmoid(-999999.0 * (CA + CA[perm] - 0.6))
    alpha16 = jnp.broadcast_to(alpha.reshape(R, 1), (R, L)).astype(jnp.float32)
    srcrow = (perm[:, None] * C + jnp.arange(C)[None, :]).reshape(R)
    srcmap8 = jnp.broadcast_to(srcrow.reshape(R, 1), (R, 8)).astype(jnp.int32)

    need = alpha.reshape(B, C) != 0.0
    pinv = jnp.argsort(perm)
    sortneed = (need | need[pinv]).reshape(R)
    sortflag16 = jnp.broadcast_to(
        sortneed.reshape(R, 1), (R, L)).astype(jnp.int32)

    mesh = plsc.VectorSubcoreMesh(core_axis_name="c", subcore_axis_name="s")

    sv = pl.kernel(
        _sort_body,
        out_type=jax.ShapeDtypeStruct((R, N), jnp.float32),
        mesh=mesh,
        compiler_params=pltpu.CompilerParams(needs_layout_passes=False),
        scratch_types=[
            pltpu.VMEM((N,), jnp.int32),
            pltpu.VMEM((N,), jnp.int32),
            pltpu.VMEM((NBINS * L,), jnp.int32),
            pltpu.VMEM((CH,), jnp.float32),
            pltpu.VMEM((L,), jnp.int32),
        ],
    )(xr, sortflag16)

    out = pl.kernel(
        _combine_body,
        out_type=jax.ShapeDtypeStruct((R, N), jnp.float32),
        mesh=mesh,
        compiler_params=pltpu.CompilerParams(needs_layout_passes=False),
        scratch_types=[
            pltpu.VMEM((N,), jnp.float32),
            pltpu.VMEM((1, N), jnp.float32),
            pltpu.VMEM((8,), jnp.int32),
            pltpu.VMEM((L,), jnp.float32),
            pltpu.VMEM((CH,), jnp.float32),
            pltpu.VMEM((CH,), jnp.float32),
            pltpu.SemaphoreType.DMA,
        ],
    )(xr, sv, alpha16, srcmap8)

    return out.reshape(B, C, W, H)


# speedup vs baseline: 8.3054x; 1.3524x over previous
import jax
import jax.numpy as jnp
from jax import lax
from jax.experimental import pallas as pl
from jax.experimental.pallas import tpu as pltpu
from jax.experimental.pallas import tpu_sc as plsc

NC, NS, L = 2, 16, 16
NW = NC * NS
B, C, W, H = 8, 96, 224, 224
N = W * H
R = B * C
RPW = R // NW
NV = N // L
NBINS = 256
CH = 6272
NCHUNK = N // CH
MIN_I32 = -2147483648


def _wid():
    return lax.axis_index("s") * NC + lax.axis_index("c")


def _to_key(v16_f32):
    b = lax.bitcast_convert_type(v16_f32, jnp.int32)
    s = jnp.right_shift(b, 31)
    return jnp.bitwise_xor(b, jnp.bitwise_or(s, jnp.int32(MIN_I32)))


def _from_key(k16_i32):
    s = jnp.right_shift(k16_i32, 31)
    m = jnp.bitwise_or(jnp.bitwise_not(s), jnp.int32(MIN_I32))
    return lax.bitcast_convert_type(jnp.bitwise_xor(k16_i32, m), jnp.float32)


def _digit(k16_i32, shift):
    ku = lax.bitcast_convert_type(k16_i32, jnp.uint32)
    return (jnp.right_shift(ku, shift) & jnp.uint32(255)).astype(jnp.int32)


def _sort_body(x_hbm, sortflag_hbm, sv_hbm, ka, kb, hist, fbuf, flagv):
    wid = _wid()
    lanes = jnp.arange(L, dtype=jnp.int32)
    ones = jnp.ones((L,), jnp.int32)
    segbase = lanes * NV

    def row(t, _):
        r = wid * RPW + t
        pltpu.sync_copy(sortflag_hbm.at[r], flagv)
        need = jnp.sum(flagv[...]) > 0

        def do_sort():
            _sort_one_row(x_hbm, sv_hbm, r, ka, kb, hist, fbuf,
                          lanes, ones, segbase)
        lax.cond(need, do_sort, lambda: None)
        return 0

    lax.fori_loop(0, RPW, row, 0)


def _sort_one_row(x_hbm, sv_hbm, r, ka, kb, hist, fbuf, lanes, ones, segbase):
        def load_chunk(ci, _):
            pltpu.sync_copy(x_hbm.at[r, pl.ds(ci * CH, CH)], fbuf)
            def conv(j, _):
                ka[pl.ds(ci * CH + j * L, L)] = _to_key(fbuf[pl.ds(j * L, L)])
                return 0
            lax.fori_loop(0, CH // L, conv, 0)
            return 0
        lax.fori_loop(0, NCHUNK, load_chunk, 0)

        for p in range(4):
            src = ka if p % 2 == 0 else kb
            dst = kb if p % 2 == 0 else ka
            shift = jnp.uint32(8 * p)

            def zero(i, _):
                hist[pl.ds(i * L, L)] = jnp.zeros((L,), jnp.int32)
                return 0
            lax.fori_loop(0, NBINS * L // L, zero, 0)

            def count(i, _):
                i2 = i * 2
                k0 = plsc.load_gather(src, [segbase + i2])
                k1 = plsc.load_gather(src, [segbase + (i2 + 1)])
                idx0 = jnp.left_shift(_digit(k0, shift), 4) + lanes
                idx1 = jnp.left_shift(_digit(k1, shift), 4) + lanes
                plsc.addupdate_scatter(hist, [idx0], ones)
                plsc.addupdate_scatter(hist, [idx1], ones)
                return 0
            lax.fori_loop(0, NV // 2, count, 0)

            def scan(i, carry):
                v = hist[pl.ds(i * L, L)]
                inc = plsc.cumsum(v)
                hist[pl.ds(i * L, L)] = inc - v + carry
                return carry + jnp.sum(v)
            lax.fori_loop(0, NBINS, scan, jnp.int32(0))

            def permute(i, _):
                i2 = i * 2
                k0 = plsc.load_gather(src, [segbase + i2])
                idx0 = jnp.left_shift(_digit(k0, shift), 4) + lanes
                pos0 = plsc.load_gather(hist, [idx0])
                plsc.store_scatter(dst, [pos0], k0)
                plsc.addupdate_scatter(hist, [idx0], ones)
                k1 = plsc.load_gather(src, [segbase + (i2 + 1)])
                idx1 = jnp.left_shift(_digit(k1, shift), 4) + lanes
                pos1 = plsc.load_gather(hist, [idx1])
                plsc.store_scatter(dst, [pos1], k1)
                plsc.addupdate_scatter(hist, [idx1], ones)
                return 0
            lax.fori_loop(0, NV // 2, permute, 0)

        def store_chunk(ci, _):
            def conv(j, _):
                fbuf[pl.ds(j * L, L)] = _from_key(ka[pl.ds(ci * CH + j * L, L)])
                return 0
            lax.fori_loop(0, CH // L, conv, 0)
            pltpu.sync_copy(fbuf, sv_hbm.at[r, pl.ds(ci * CH, CH)])
            return 0
        lax.fori_loop(0, NCHUNK, store_chunk, 0)


def _combine_body(x_hbm, sv_hbm, alpha_hbm, srcmap_hbm, out_hbm,
                  svb, svp, srcidx, av, xc, oc, sem):
    wid = _wid()
    zeros16 = jnp.zeros((L,), jnp.int32)

    def row(t, _):
        r = wid * RPW + t
        pltpu.sync_copy(alpha_hbm.at[r], av)
        a16 = av[...]
        active = jnp.sum(jnp.where(a16 != 0.0, jnp.int32(1), jnp.int32(0))) > 0

        def match():
            pltpu.sync_copy(sv_hbm.at[r], svb)
            pltpu.sync_copy(srcmap_hbm.at[r], srcidx)
            pltpu.async_copy(sv_hbm.at[srcidx.at[pl.ds(0, 1)]], svp, sem).wait()
            alpha = av[...]

            def chunk(ci, _):
                pltpu.sync_copy(x_hbm.at[r, pl.ds(ci * CH, CH)], xc)

                @plsc.parallel_loop(0, CH // L, unroll=2)
                def _elem(j):
                    xv = xc[pl.ds(j * L, L)]
                    rk = jnp.zeros((L,), jnp.int32)
                    for bit in (32768, 16384, 8192, 4096, 2048, 1024, 512,
                                256, 128, 64, 32, 16, 8, 4, 2, 1):
                        nr = rk + bit
                        im = jnp.minimum(nr, jnp.int32(N)) - 1
                        v = plsc.load_gather(svb, [im])
                        rk = jnp.where(v < xv, nr, rk)
                    m = plsc.load_gather(svp, [zeros16, rk])
                    oc[pl.ds(j * L, L)] = xv + alpha * (m - xv)
                pltpu.sync_copy(oc, out_hbm.at[r, pl.ds(ci * CH, CH)])
                return 0
            lax.fori_loop(0, NCHUNK, chunk, 0)

        def copy():
            def chunk(ci, _):
                pltpu.sync_copy(x_hbm.at[r, pl.ds(ci * CH, CH)], xc)
                pltpu.sync_copy(xc, out_hbm.at[r, pl.ds(ci * CH, CH)])
                return 0
            lax.fori_loop(0, NCHUNK, chunk, 0)

        lax.cond(active, match, copy)
        return 0

    lax.fori_loop(0, RPW, row, 0)


def kernel(CA, x):
    xr = x.reshape(R, N)

    perm = jnp.arange(B - 1, -1, -1)
    pk1, pk2 = jax.random.split(jax.random.key(42))
    perm_b = perm[: B // 2][jax.random.permutation(pk1, B // 2)]
    perm_a = perm[B // 2:][jax.random.permutation(pk2, B // 2)]
    perm = jnp.concatenate([perm_b, perm_a], axis=0)

    alpha = jax.nn.sigmoid(-999999.0 * (CA + CA[perm] - 0.6))
    alpha16 = jnp.broadcast_to(alpha.reshape(R, 1), (R, L)).astype(jnp.float32)
    srcrow = (perm[:, None] * C + jnp.arange(C)[None, :]).reshape(R)
    srcmap8 = jnp.broadcast_to(srcrow.reshape(R, 1), (R, 8)).astype(jnp.int32)

    need = alpha.reshape(B, C) != 0.0
    pinv = jnp.argsort(perm)
    sortneed = (need | need[pinv]).reshape(R)
    sortflag16 = jnp.broadcast_to(
        sortneed.reshape(R, 1), (R, L)).astype(jnp.int32)

    mesh = plsc.VectorSubcoreMesh(core_axis_name="c", subcore_axis_name="s")

    sv = pl.kernel(
        _sort_body,
        out_type=jax.ShapeDtypeStruct((R, N), jnp.float32),
        mesh=mesh,
        compiler_params=pltpu.CompilerParams(needs_layout_passes=False),
        scratch_types=[
            pltpu.VMEM((N,), jnp.int32),
            pltpu.VMEM((N,), jnp.int32),
            pltpu.VMEM((NBINS * L,), jnp.int32),
            pltpu.VMEM((CH,), jnp.float32),
            pltpu.VMEM((L,), jnp.int32),
        ],
    )(xr, sortflag16)

    out = pl.kernel(
        _combine_body,
        out_type=jax.ShapeDtypeStruct((R, N), jnp.float32),
        mesh=mesh,
        compiler_params=pltpu.CompilerParams(needs_layout_passes=False),
        scratch_types=[
            pltpu.VMEM((N,), jnp.float32),
            pltpu.VMEM((1, N), jnp.float32),
            pltpu.VMEM((8,), jnp.int32),
            pltpu.VMEM((L,), jnp.float32),
            pltpu.VMEM((CH,), jnp.float32),
            pltpu.VMEM((CH,), jnp.float32),
            pltpu.SemaphoreType.DMA,
        ],
    )(xr, sv, alpha16, srcmap8)

    return out.reshape(B, C, W, H)


# speedup vs baseline: 8.8850x; 1.0698x over previous
import jax
import jax.numpy as jnp
from jax import lax
from jax.experimental import pallas as pl
from jax.experimental.pallas import tpu as pltpu
from jax.experimental.pallas import tpu_sc as plsc

NC, NS, L = 2, 16, 16
NW = NC * NS
B, C, W, H = 8, 96, 224, 224
N = W * H
R = B * C
RPW = R // NW
NV = N // L
NBINS = 256
CH = 6272
NCHUNK = N // CH
MIN_I32 = -2147483648


def _wid():
    return lax.axis_index("s") * NC + lax.axis_index("c")


def _to_key(v16_f32):
    b = lax.bitcast_convert_type(v16_f32, jnp.int32)
    s = jnp.right_shift(b, 31)
    return jnp.bitwise_xor(b, jnp.bitwise_or(s, jnp.int32(MIN_I32)))


def _from_key(k16_i32):
    s = jnp.right_shift(k16_i32, 31)
    m = jnp.bitwise_or(jnp.bitwise_not(s), jnp.int32(MIN_I32))
    return lax.bitcast_convert_type(jnp.bitwise_xor(k16_i32, m), jnp.float32)


def _digit(k16_i32, shift):
    ku = lax.bitcast_convert_type(k16_i32, jnp.uint32)
    return (jnp.right_shift(ku, shift) & jnp.uint32(255)).astype(jnp.int32)


def _sort_body(x_hbm, sortflag_hbm, sv_hbm, ka, kb, hist, fbuf, flagv):
    wid = _wid()
    lanes = jnp.arange(L, dtype=jnp.int32)
    ones = jnp.ones((L,), jnp.int32)
    segbase = lanes * NV

    def row(t, _):
        r = wid * RPW + t
        pltpu.sync_copy(sortflag_hbm.at[r], flagv)
        need = jnp.sum(flagv[...]) > 0

        def do_sort():
            _sort_one_row(x_hbm, sv_hbm, r, ka, kb, hist, fbuf,
                          lanes, ones, segbase)
        lax.cond(need, do_sort, lambda: None)
        return 0

    lax.fori_loop(0, RPW, row, 0)


def _sort_one_row(x_hbm, sv_hbm, r, ka, kb, hist, fbuf, lanes, ones, segbase):
        def load_chunk(ci, _):
            pltpu.sync_copy(x_hbm.at[r, pl.ds(ci * CH, CH)], fbuf)

            @plsc.parallel_loop(0, CH // L, unroll=4)
            def _conv(j):
                ka[pl.ds(ci * CH + j * L, L)] = _to_key(fbuf[pl.ds(j * L, L)])
            return 0
        lax.fori_loop(0, NCHUNK, load_chunk, 0)

        for p in range(4):
            src = ka if p % 2 == 0 else kb
            dst = kb if p % 2 == 0 else ka
            shift = jnp.uint32(8 * p)

            def zero(i, _):
                hist[pl.ds(i * L, L)] = jnp.zeros((L,), jnp.int32)
                return 0
            lax.fori_loop(0, NBINS * L // L, zero, 0)

            def count(i, _):
                i4 = i * 4
                ks = [plsc.load_gather(src, [segbase + (i4 + u)])
                      for u in range(4)]
                idxs = [jnp.left_shift(_digit(k, shift), 4) + lanes
                        for k in ks]
                for idx in idxs:
                    plsc.addupdate_scatter(hist, [idx], ones)
                return 0
            lax.fori_loop(0, NV // 4, count, 0)

            def scan(i, carry):
                v = hist[pl.ds(i * L, L)]
                inc = plsc.cumsum(v)
                hist[pl.ds(i * L, L)] = inc - v + carry
                return carry + jnp.sum(v)
            lax.fori_loop(0, NBINS, scan, jnp.int32(0))

            def permute(i, _):
                i4 = i * 4
                for u in range(4):
                    k = plsc.load_gather(src, [segbase + (i4 + u)])
                    idx = jnp.left_shift(_digit(k, shift), 4) + lanes
                    pos = plsc.load_gather(hist, [idx])
                    plsc.store_scatter(dst, [pos], k)
                    plsc.addupdate_scatter(hist, [idx], ones)
                return 0
            lax.fori_loop(0, NV // 4, permute, 0)

        def store_chunk(ci, _):
            @plsc.parallel_loop(0, CH // L, unroll=4)
            def _conv(j):
                fbuf[pl.ds(j * L, L)] = _from_key(ka[pl.ds(ci * CH + j * L, L)])
            pltpu.sync_copy(fbuf, sv_hbm.at[r, pl.ds(ci * CH, CH)])
            return 0
        lax.fori_loop(0, NCHUNK, store_chunk, 0)


def _combine_body(x_hbm, sv_hbm, alpha_hbm, srcmap_hbm, out_hbm,
                  svb, svp, srcidx, av, xc, oc, sem):
    wid = _wid()
    zeros16 = jnp.zeros((L,), jnp.int32)

    def row(t, _):
        r = wid * RPW + t
        pltpu.sync_copy(alpha_hbm.at[r], av)
        a16 = av[...]
        active = jnp.sum(jnp.where(a16 != 0.0, jnp.int32(1), jnp.int32(0))) > 0

        def match():
            pltpu.sync_copy(sv_hbm.at[r], svb)
            pltpu.sync_copy(srcmap_hbm.at[r], srcidx)
            pltpu.async_copy(sv_hbm.at[srcidx.at[pl.ds(0, 1)]], svp, sem).wait()
            alpha = av[...]

            def chunk(ci, _):
                pltpu.sync_copy(x_hbm.at[r, pl.ds(ci * CH, CH)], xc)

                @plsc.parallel_loop(0, CH // L, unroll=4)
                def _elem(j):
                    xv = xc[pl.ds(j * L, L)]
                    rk = jnp.zeros((L,), jnp.int32)
                    for bit in (32768, 16384, 8192, 4096, 2048, 1024, 512,
                                256, 128, 64, 32, 16, 8, 4, 2, 1):
                        nr = rk + bit
                        im = jnp.minimum(nr, jnp.int32(N)) - 1
                        v = plsc.load_gather(svb, [im])
                        rk = jnp.where(v < xv, nr, rk)
                    m = plsc.load_gather(svp, [zeros16, rk])
                    oc[pl.ds(j * L, L)] = xv + alpha * (m - xv)
                pltpu.sync_copy(oc, out_hbm.at[r, pl.ds(ci * CH, CH)])
                return 0
            lax.fori_loop(0, NCHUNK, chunk, 0)

        def copy():
            def chunk(ci, _):
                pltpu.sync_copy(x_hbm.at[r, pl.ds(ci * CH, CH)], xc)
                pltpu.sync_copy(xc, out_hbm.at[r, pl.ds(ci * CH, CH)])
                return 0
            lax.fori_loop(0, NCHUNK, chunk, 0)

        lax.cond(active, match, copy)
        return 0

    lax.fori_loop(0, RPW, row, 0)


def kernel(CA, x):
    xr = x.reshape(R, N)

    perm = jnp.arange(B - 1, -1, -1)
    pk1, pk2 = jax.random.split(jax.random.key(42))
    perm_b = perm[: B // 2][jax.random.permutation(pk1, B // 2)]
    perm_a = perm[B // 2:][jax.random.permutation(pk2, B // 2)]
    perm = jnp.concatenate([perm_b, perm_a], axis=0)

    alpha = jax.nn.sigmoid(-999999.0 * (CA + CA[perm] - 0.6))
    alpha16 = jnp.broadcast_to(alpha.reshape(R, 1), (R, L)).astype(jnp.float32)
    srcrow = (perm[:, None] * C + jnp.arange(C)[None, :]).reshape(R)
    srcmap8 = jnp.broadcast_to(srcrow.reshape(R, 1), (R, 8)).astype(jnp.int32)

    need = alpha.reshape(B, C) != 0.0
    pinv = jnp.argsort(perm)
    sortneed = (need | need[pinv]).reshape(R)
    sortflag16 = jnp.broadcast_to(
        sortneed.reshape(R, 1), (R, L)).astype(jnp.int32)

    mesh = plsc.VectorSubcoreMesh(core_axis_name="c", subcore_axis_name="s")

    sv = pl.kernel(
        _sort_body,
        out_type=jax.ShapeDtypeStruct((R, N), jnp.float32),
        mesh=mesh,
        compiler_params=pltpu.CompilerParams(needs_layout_passes=False),
        scratch_types=[
            pltpu.VMEM((N,), jnp.int32),
            pltpu.VMEM((N,), jnp.int32),
            pltpu.VMEM((NBINS * L,), jnp.int32),
            pltpu.VMEM((CH,), jnp.float32),
            pltpu.VMEM((L,), jnp.int32),
        ],
    )(xr, sortflag16)

    out = pl.kernel(
        _combine_body,
        out_type=jax.ShapeDtypeStruct((R, N), jnp.float32),
        mesh=mesh,
        compiler_params=pltpu.CompilerParams(needs_layout_passes=False),
        scratch_types=[
            pltpu.VMEM((N,), jnp.float32),
            pltpu.VMEM((1, N), jnp.float32),
            pltpu.VMEM((8,), jnp.int32),
            pltpu.VMEM((L,), jnp.float32),
            pltpu.VMEM((CH,), jnp.float32),
            pltpu.VMEM((CH,), jnp.float32),
            pltpu.SemaphoreType.DMA,
        ],
    )(xr, sv, alpha16, srcmap8)

    return out.reshape(B, C, W, H)


# speedup vs baseline: 11.4258x; 1.2860x over previous
import jax
import jax.numpy as jnp
from jax import lax
from jax.experimental import pallas as pl
from jax.experimental.pallas import tpu as pltpu
from jax.experimental.pallas import tpu_sc as plsc

NC, NS, L = 2, 16, 16
NW = NC * NS
B, C, W, H = 8, 96, 224, 224
N = W * H
R = B * C
RPW = R // NW
NV = N // L
NBINS = 256
CH = 6272
NCHUNK = N // CH
MIN_I32 = -2147483648


def _wid():
    return lax.axis_index("s") * NC + lax.axis_index("c")


def _to_key(v16_f32):
    b = lax.bitcast_convert_type(v16_f32, jnp.int32)
    s = jnp.right_shift(b, 31)
    return jnp.bitwise_xor(b, jnp.bitwise_or(s, jnp.int32(MIN_I32)))


def _from_key(k16_i32):
    s = jnp.right_shift(k16_i32, 31)
    m = jnp.bitwise_or(jnp.bitwise_not(s), jnp.int32(MIN_I32))
    return lax.bitcast_convert_type(jnp.bitwise_xor(k16_i32, m), jnp.float32)


def _digit(k16_i32, shift):
    ku = lax.bitcast_convert_type(k16_i32, jnp.uint32)
    return (jnp.right_shift(ku, shift) & jnp.uint32(255)).astype(jnp.int32)


def _sort_body(x3_hbm, wl_hbm, k1_hbm, sv_hbm, ka, kb, hist, fbuf, wlv, k1v):
    wid = _wid()
    lanes = jnp.arange(L, dtype=jnp.int32)
    ones = jnp.ones((L,), jnp.int32)
    segbase = lanes * NV

    pltpu.sync_copy(wl_hbm.at[wid], wlv)
    pltpu.sync_copy(k1_hbm, k1v)
    k1 = jnp.max(k1v[...])

    def slot(t, _):
        pos = wid + t * NW

        def do_sort():
            _sort_one_row(x3_hbm, sv_hbm, wlv, t, pos, ka, kb, hist, fbuf,
                          lanes, ones, segbase)
        lax.cond(pos < k1, do_sort, lambda: None)
        return 0

    lax.fori_loop(0, RPW, slot, 0)


def _sort_one_row(x3_hbm, sv_hbm, wlv, t, pos, ka, kb, hist, fbuf,
                  lanes, ones, segbase):
        def load_chunk(ci, _):
            pltpu.sync_copy(x3_hbm.at[wlv.at[t, pl.ds(ci, 1)]], fbuf)

            @plsc.parallel_loop(0, CH // L, unroll=4)
            def _conv(j):
                ka[pl.ds(ci * CH + j * L, L)] = _to_key(fbuf[0, pl.ds(j * L, L)])
            return 0
        lax.fori_loop(0, NCHUNK, load_chunk, 0)

        for p in range(4):
            src = ka if p % 2 == 0 else kb
            dst = kb if p % 2 == 0 else ka
            shift = jnp.uint32(8 * p)

            def zero(i, _):
                hist[pl.ds(i * L, L)] = jnp.zeros((L,), jnp.int32)
                return 0
            lax.fori_loop(0, NBINS * L // L, zero, 0)

            def count(i, _):
                i4 = i * 4
                ks = [plsc.load_gather(src, [segbase + (i4 + u)])
                      for u in range(4)]
                idxs = [jnp.left_shift(_digit(k, shift), 4) + lanes
                        for k in ks]
                for idx in idxs:
                    plsc.addupdate_scatter(hist, [idx], ones)
                return 0
            lax.fori_loop(0, NV // 4, count, 0)

            def scan(i, carry):
                v = hist[pl.ds(i * L, L)]
                inc = plsc.cumsum(v)
                hist[pl.ds(i * L, L)] = inc - v + carry
                return carry + jnp.sum(v)
            lax.fori_loop(0, NBINS, scan, jnp.int32(0))

            def permute(i, _):
                i4 = i * 4
                for u in range(4):
                    k = plsc.load_gather(src, [segbase + (i4 + u)])
                    idx = jnp.left_shift(_digit(k, shift), 4) + lanes
                    pos = plsc.load_gather(hist, [idx])
                    plsc.store_scatter(dst, [pos], k)
                    plsc.addupdate_scatter(hist, [idx], ones)
                return 0
            lax.fori_loop(0, NV // 4, permute, 0)

        def store_chunk(ci, _):
            @plsc.parallel_loop(0, CH // L, unroll=4)
            def _conv(j):
                fbuf[0, pl.ds(j * L, L)] = _from_key(
                    ka[pl.ds(ci * CH + j * L, L)])
            pltpu.sync_copy(fbuf, sv_hbm.at[pl.ds(pos, 1), pl.ds(ci * CH, CH)])
            return 0
        lax.fori_loop(0, NCHUNK, store_chunk, 0)


def _combine_body(x_hbm, sv_hbm, alpha_hbm, posb_hbm, posp_hbm, out_hbm,
                  svb, svp, bidx, pidx, av, xc, oc, sem):
    wid = _wid()
    zeros16 = jnp.zeros((L,), jnp.int32)

    def row(t, _):
        r = wid * RPW + t
        pltpu.sync_copy(alpha_hbm.at[r], av)
        a16 = av[...]
        active = jnp.sum(jnp.where(a16 != 0.0, jnp.int32(1), jnp.int32(0))) > 0

        def match():
            pltpu.sync_copy(posb_hbm.at[r], bidx)
            pltpu.sync_copy(posp_hbm.at[r], pidx)
            pltpu.async_copy(sv_hbm.at[bidx.at[pl.ds(0, 1)]], svb, sem).wait()
            pltpu.async_copy(sv_hbm.at[pidx.at[pl.ds(0, 1)]], svp, sem).wait()
            alpha = av[...]

            def chunk(ci, _):
                pltpu.sync_copy(x_hbm.at[r, pl.ds(ci * CH, CH)], xc)

                @plsc.parallel_loop(0, CH // L, unroll=4)
                def _elem(j):
                    xv = xc[pl.ds(j * L, L)]
                    rk = jnp.zeros((L,), jnp.int32)
                    for bit in (32768, 16384, 8192, 4096, 2048, 1024, 512,
                                256, 128, 64, 32, 16, 8, 4, 2, 1):
                        nr = rk + bit
                        im = jnp.minimum(nr, jnp.int32(N)) - 1
                        v = plsc.load_gather(svb, [zeros16, im])
                        rk = jnp.where(v < xv, nr, rk)
                    m = plsc.load_gather(svp, [zeros16, rk])
                    oc[pl.ds(j * L, L)] = xv + alpha * (m - xv)
                pltpu.sync_copy(oc, out_hbm.at[r, pl.ds(ci * CH, CH)])
                return 0
            lax.fori_loop(0, NCHUNK, chunk, 0)

        def copy():
            def chunk(ci, _):
                pltpu.sync_copy(x_hbm.at[r, pl.ds(ci * CH, CH)], xc)
                pltpu.sync_copy(xc, out_hbm.at[r, pl.ds(ci * CH, CH)])
                return 0
            lax.fori_loop(0, NCHUNK, chunk, 0)

        lax.cond(active, match, copy)
        return 0

    lax.fori_loop(0, RPW, row, 0)


def kernel(CA, x):
    xr = x.reshape(R, N)

    perm = jnp.arange(B - 1, -1, -1)
    pk1, pk2 = jax.random.split(jax.random.key(42))
    perm_b = perm[: B // 2][jax.random.permutation(pk1, B // 2)]
    perm_a = perm[B // 2:][jax.random.permutation(pk2, B // 2)]
    perm = jnp.concatenate([perm_b, perm_a], axis=0)

    alpha = jax.nn.sigmoid(-999999.0 * (CA + CA[perm] - 0.6))
    alpha16 = jnp.broadcast_to(alpha.reshape(R, 1), (R, L)).astype(jnp.float32)
    srcrow = (perm[:, None] * C + jnp.arange(C)[None, :]).reshape(R)
    srcmap8 = jnp.broadcast_to(srcrow.reshape(R, 1), (R, 8)).astype(jnp.int32)

    need = alpha.reshape(B, C) != 0.0
    pinv = jnp.argsort(perm)
    sortneed = (need | need[pinv]).reshape(R)

    orderA = jnp.argsort(1 - sortneed.astype(jnp.int32), stable=True)
    k1 = jnp.sum(sortneed.astype(jnp.int32)).astype(jnp.int32)
    k1v = jnp.full((L,), k1, jnp.int32)
    posA = jnp.argsort(orderA).astype(jnp.int32)
    slots = (jnp.arange(NW)[:, None] + jnp.arange(RPW)[None, :] * NW)
    wlA8 = (orderA[slots][..., None] * 8
            + jnp.arange(8)[None, None, :]).astype(jnp.int32)
    posb8 = jnp.broadcast_to(posA.reshape(R, 1), (R, 8)).astype(jnp.int32)
    posp8 = jnp.broadcast_to(posA[srcrow].reshape(R, 1), (R, 8)).astype(jnp.int32)

    mesh = plsc.VectorSubcoreMesh(core_axis_name="c", subcore_axis_name="s")

    sv = pl.kernel(
        _sort_body,
        out_type=jax.ShapeDtypeStruct((R, N), jnp.float32),
        mesh=mesh,
        compiler_params=pltpu.CompilerParams(needs_layout_passes=False),
        scratch_types=[
            pltpu.VMEM((N,), jnp.int32),
            pltpu.VMEM((N,), jnp.int32),
            pltpu.VMEM((NBINS * L,), jnp.int32),
            pltpu.VMEM((1, CH), jnp.float32),
            pltpu.VMEM((RPW, 8), jnp.int32),
            pltpu.VMEM((L,), jnp.int32),
        ],
    )(xr.reshape(R * NCHUNK, CH), wlA8, k1v)

    out = pl.kernel(
        _combine_body,
        out_type=jax.ShapeDtypeStruct((R, N), jnp.float32),
        mesh=mesh,
        compiler_params=pltpu.CompilerParams(needs_layout_passes=False),
        scratch_types=[
            pltpu.VMEM((1, N), jnp.float32),
            pltpu.VMEM((1, N), jnp.float32),
            pltpu.VMEM((8,), jnp.int32),
            pltpu.VMEM((8,), jnp.int32),
            pltpu.VMEM((L,), jnp.float32),
            pltpu.VMEM((CH,), jnp.float32),
            pltpu.VMEM((CH,), jnp.float32),
            pltpu.SemaphoreType.DMA,
        ],
    )(xr, sv, alpha16, posb8, posp8)

    return out.reshape(B, C, W, H)


# speedup vs baseline: 14.1498x; 1.2384x over previous
import jax
import jax.numpy as jnp
from jax import lax
from jax.experimental import pallas as pl
from jax.experimental.pallas import tpu as pltpu
from jax.experimental.pallas import tpu_sc as plsc

NC, NS, L = 2, 16, 16
NW = NC * NS
B, C, W, H = 8, 96, 224, 224
N = W * H
R = B * C
RPW = R // NW
NV = N // L
NBINS = 256
CH = 6272
NCHUNK = N // CH
MIN_I32 = -2147483648


def _wid():
    return lax.axis_index("s") * NC + lax.axis_index("c")


def _to_key(v16_f32):
    b = lax.bitcast_convert_type(v16_f32, jnp.int32)
    s = jnp.right_shift(b, 31)
    return jnp.bitwise_xor(b, jnp.bitwise_or(s, jnp.int32(MIN_I32)))


def _from_key(k16_i32):
    s = jnp.right_shift(k16_i32, 31)
    m = jnp.bitwise_or(jnp.bitwise_not(s), jnp.int32(MIN_I32))
    return lax.bitcast_convert_type(jnp.bitwise_xor(k16_i32, m), jnp.float32)


def _digit(k16_i32, shift):
    ku = lax.bitcast_convert_type(k16_i32, jnp.uint32)
    return (jnp.right_shift(ku, shift) & jnp.uint32(255)).astype(jnp.int32)


def _sort_body(x3_hbm, wl_hbm, k1_hbm, sv_hbm, ka, kb, hist, fbuf, wlv, k1v):
    wid = _wid()
    lanes = jnp.arange(L, dtype=jnp.int32)
    ones = jnp.ones((L,), jnp.int32)
    segbase = lanes * NV

    pltpu.sync_copy(wl_hbm.at[wid], wlv)
    pltpu.sync_copy(k1_hbm, k1v)
    k1 = jnp.max(k1v[...])

    def slot(t, _):
        pos = wid + t * NW

        def do_sort():
            _sort_one_row(x3_hbm, sv_hbm, wlv, t, pos, ka, kb, hist, fbuf,
                          lanes, ones, segbase)
        lax.cond(pos < k1, do_sort, lambda: None)
        return 0

    lax.fori_loop(0, RPW, slot, 0)


def _sort_one_row(x3_hbm, sv_hbm, wlv, t, pos, ka, kb, hist, fbuf,
                  lanes, ones, segbase):
        def load_chunk(ci, _):
            pltpu.sync_copy(x3_hbm.at[wlv.at[t, pl.ds(ci, 1)]], fbuf)

            @plsc.parallel_loop(0, CH // L, unroll=4)
            def _conv(j):
                ka[pl.ds(ci * CH + j * L, L)] = _to_key(fbuf[0, pl.ds(j * L, L)])
            return 0
        lax.fori_loop(0, NCHUNK, load_chunk, 0)

        for p in range(4):
            src = ka if p % 2 == 0 else kb
            dst = kb if p % 2 == 0 else ka
            shift = jnp.uint32(8 * p)

            def zero(i, _):
                hist[pl.ds(i * L, L)] = jnp.zeros((L,), jnp.int32)
                return 0
            lax.fori_loop(0, NBINS * L // L, zero, 0)

            def count(i, _):
                i4 = i * 4
                ks = [plsc.load_gather(src, [segbase + (i4 + u)])
                      for u in range(4)]
                idxs = [jnp.left_shift(_digit(k, shift), 4) + lanes
                        for k in ks]
                for idx in idxs:
                    plsc.addupdate_scatter(hist, [idx], ones)
                return 0
            lax.fori_loop(0, NV // 4, count, 0)

            def scan(i, carry):
                v = hist[pl.ds(i * L, L)]
                inc = plsc.cumsum(v)
                hist[pl.ds(i * L, L)] = inc - v + carry
                return carry + jnp.sum(v)
            lax.fori_loop(0, NBINS, scan, jnp.int32(0))

            def permute(i, _):
                i4 = i * 4
                for u in range(4):
                    k = plsc.load_gather(src, [segbase + (i4 + u)])
                    idx = jnp.left_shift(_digit(k, shift), 4) + lanes
                    pos = plsc.load_gather(hist, [idx])
                    plsc.store_scatter(dst, [pos], k)
                    plsc.addupdate_scatter(hist, [idx], ones)
                return 0
            lax.fori_loop(0, NV // 4, permute, 0)

        def store_chunk(ci, _):
            @plsc.parallel_loop(0, CH // L, unroll=4)
            def _conv(j):
                fbuf[0, pl.ds(j * L, L)] = _from_key(
                    ka[pl.ds(ci * CH + j * L, L)])
            pltpu.sync_copy(fbuf, sv_hbm.at[pl.ds(pos, 1), pl.ds(ci * CH, CH)])
            return 0
        lax.fori_loop(0, NCHUNK, store_chunk, 0)


def _match_body(x3_hbm, sv_hbm, alphaS_hbm, svbS_hbm, svpS_hbm, wlx_hbm,
                k2_hbm, mout_hbm, svb, svp, bidx, pidx, av, wlv, k2v, xc, oc,
                sem):
    wid = _wid()
    zeros16 = jnp.zeros((L,), jnp.int32)
    pltpu.sync_copy(wlx_hbm.at[wid], wlv)
    pltpu.sync_copy(k2_hbm, k2v)
    k2 = jnp.max(k2v[...])

    def slot(t, _):
        pos = wid + t * NW

        def match():
            pltpu.sync_copy(alphaS_hbm.at[pos], av)
            pltpu.sync_copy(svbS_hbm.at[pos], bidx)
            pltpu.sync_copy(svpS_hbm.at[pos], pidx)
            pltpu.async_copy(sv_hbm.at[bidx.at[pl.ds(0, 1)]], svb, sem).wait()
            pltpu.async_copy(sv_hbm.at[pidx.at[pl.ds(0, 1)]], svp, sem).wait()
            alpha = av[...]

            def chunk(ci, _):
                pltpu.sync_copy(x3_hbm.at[wlv.at[t, pl.ds(ci, 1)]], xc)

                @plsc.parallel_loop(0, CH // L, unroll=4)
                def _elem(j):
                    xv = xc[0, pl.ds(j * L, L)]
                    rk = jnp.zeros((L,), jnp.int32)
                    for bit in (32768, 16384, 8192, 4096, 2048, 1024, 512,
                                256, 128, 64, 32, 16, 8, 4, 2, 1):
                        nr = rk + bit
                        im = jnp.minimum(nr, jnp.int32(N)) - 1
                        v = plsc.load_gather(svb, [zeros16, im])
                        rk = jnp.where(v < xv, nr, rk)
                    m = plsc.load_gather(svp, [zeros16, rk])
                    oc[pl.ds(j * L, L)] = xv + alpha * (m - xv)
                pltpu.sync_copy(oc, mout_hbm.at[pos, pl.ds(ci * CH, CH)])
                return 0
            lax.fori_loop(0, NCHUNK, chunk, 0)

        lax.cond(pos < k2, match, lambda: None)
        return 0

    lax.fori_loop(0, RPW, slot, 0)


def _distribute_body(x_hbm, m3_hbm, alpha_hbm, posm_hbm, out_hbm,
                     av, pm, xc1, xc, sem):
    wid = _wid()

    def row(t, _):
        r = wid * RPW + t
        pltpu.sync_copy(alpha_hbm.at[r], av)
        a16 = av[...]
        active = jnp.sum(jnp.where(a16 != 0.0, jnp.int32(1), jnp.int32(0))) > 0

        def dist():
            pltpu.sync_copy(posm_hbm.at[r], pm)

            def chunk(ci, _):
                pltpu.async_copy(m3_hbm.at[pm.at[ci]], xc1, sem).wait()
                pltpu.sync_copy(xc1, out_hbm.at[pl.ds(r, 1),
                                                pl.ds(ci * CH, CH)])
                return 0
            lax.fori_loop(0, NCHUNK, chunk, 0)

        def copy():
            def chunk(ci, _):
                pltpu.sync_copy(x_hbm.at[r, pl.ds(ci * CH, CH)], xc)
                pltpu.sync_copy(xc, out_hbm.at[r, pl.ds(ci * CH, CH)])
                return 0
            lax.fori_loop(0, NCHUNK, chunk, 0)

        lax.cond(active, dist, copy)
        return 0

    lax.fori_loop(0, RPW, row, 0)


def kernel(CA, x):
    xr = x.reshape(R, N)

    perm = jnp.arange(B - 1, -1, -1)
    pk1, pk2 = jax.random.split(jax.random.key(42))
    perm_b = perm[: B // 2][jax.random.permutation(pk1, B // 2)]
    perm_a = perm[B // 2:][jax.random.permutation(pk2, B // 2)]
    perm = jnp.concatenate([perm_b, perm_a], axis=0)

    alpha = jax.nn.sigmoid(-999999.0 * (CA + CA[perm] - 0.6))
    alpha16 = jnp.broadcast_to(alpha.reshape(R, 1), (R, L)).astype(jnp.float32)
    srcrow = (perm[:, None] * C + jnp.arange(C)[None, :]).reshape(R)

    need = alpha.reshape(B, C) != 0.0
    pinv = jnp.argsort(perm)
    sortneed = (need | need[pinv]).reshape(R)

    orderA = jnp.argsort(1 - sortneed.astype(jnp.int32), stable=True)
    k1 = jnp.sum(sortneed.astype(jnp.int32)).astype(jnp.int32)
    k1v = jnp.full((L,), k1, jnp.int32)
    posA = jnp.argsort(orderA).astype(jnp.int32)
    slots = (jnp.arange(NW)[:, None] + jnp.arange(RPW)[None, :] * NW)
    wlA8 = (orderA[slots][..., None] * 8
            + jnp.arange(8)[None, None, :]).astype(jnp.int32)
    posb8 = jnp.broadcast_to(posA.reshape(R, 1), (R, 8)).astype(jnp.int32)
    posp8 = jnp.broadcast_to(posA[srcrow].reshape(R, 1), (R, 8)).astype(jnp.int32)

    needR = need.reshape(R)
    orderB = jnp.argsort(1 - needR.astype(jnp.int32), stable=True)
    k2 = jnp.sum(needR.astype(jnp.int32)).astype(jnp.int32)
    k2v = jnp.full((L,), k2, jnp.int32)
    posB = jnp.argsort(orderB).astype(jnp.int32)
    wlBx8 = (orderB[slots][..., None] * NCHUNK
             + jnp.arange(NCHUNK)[None, None, :]).astype(jnp.int32)
    alphaS = alpha16[orderB]
    svbS8 = posb8[orderB]
    svpS8 = posp8[orderB]
    posm8 = (posB.reshape(R, 1) * NCHUNK
             + jnp.arange(NCHUNK)[None, :]).astype(jnp.int32).reshape(R, NCHUNK, 1)

    mesh = plsc.VectorSubcoreMesh(core_axis_name="c", subcore_axis_name="s")

    sv = pl.kernel(
        _sort_body,
        out_type=jax.ShapeDtypeStruct((R, N), jnp.float32),
        mesh=mesh,
        compiler_params=pltpu.CompilerParams(needs_layout_passes=False),
        scratch_types=[
            pltpu.VMEM((N,), jnp.int32),
            pltpu.VMEM((N,), jnp.int32),
            pltpu.VMEM((NBINS * L,), jnp.int32),
            pltpu.VMEM((1, CH), jnp.float32),
            pltpu.VMEM((RPW, 8), jnp.int32),
            pltpu.VMEM((L,), jnp.int32),
        ],
    )(xr.reshape(R * NCHUNK, CH), wlA8, k1v)

    mout = pl.kernel(
        _match_body,
        out_type=jax.ShapeDtypeStruct((R, N), jnp.float32),
        mesh=mesh,
        compiler_params=pltpu.CompilerParams(needs_layout_passes=False),
        scratch_types=[
            pltpu.VMEM((1, N), jnp.float32),
            pltpu.VMEM((1, N), jnp.float32),
            pltpu.VMEM((8,), jnp.int32),
            pltpu.VMEM((8,), jnp.int32),
            pltpu.VMEM((L,), jnp.float32),
            pltpu.VMEM((RPW, 8), jnp.int32),
            pltpu.VMEM((L,), jnp.int32),
            pltpu.VMEM((1, CH), jnp.float32),
            pltpu.VMEM((CH,), jnp.float32),
            pltpu.SemaphoreType.DMA,
        ],
    )(xr.reshape(R * NCHUNK, CH), sv, alphaS, svbS8, svpS8, wlBx8, k2v)

    out = pl.kernel(
        _distribute_body,
        out_type=jax.ShapeDtypeStruct((R, N), jnp.float32),
        mesh=mesh,
        compiler_params=pltpu.CompilerParams(needs_layout_passes=False),
        scratch_types=[
            pltpu.VMEM((L,), jnp.float32),
            pltpu.VMEM((NCHUNK, 1), jnp.int32),
            pltpu.VMEM((1, CH), jnp.float32),
            pltpu.VMEM((CH,), jnp.float32),
            pltpu.SemaphoreType.DMA,
        ],
    )(xr, mout.reshape(R * NCHUNK, CH), alpha16, posm8)

    return out.reshape(B, C, W, H)


# speedup vs baseline: 14.9892x; 1.0593x over previous
import jax
import jax.numpy as jnp
from jax import lax
from jax.experimental import pallas as pl
from jax.experimental.pallas import tpu as pltpu
from jax.experimental.pallas import tpu_sc as plsc

NC, NS, L = 2, 16, 16
NW = NC * NS
B, C, W, H = 8, 96, 224, 224
N = W * H
R = B * C
RPW = R // NW
NV = N // L
NBINS = 256
CH = 6272
NCHUNK = N // CH
MIN_I32 = -2147483648


def _wid():
    return lax.axis_index("s") * NC + lax.axis_index("c")


def _to_key(v16_f32):
    b = lax.bitcast_convert_type(v16_f32, jnp.int32)
    s = jnp.right_shift(b, 31)
    return jnp.bitwise_xor(b, jnp.bitwise_or(s, jnp.int32(MIN_I32)))


def _from_key(k16_i32):
    s = jnp.right_shift(k16_i32, 31)
    m = jnp.bitwise_or(jnp.bitwise_not(s), jnp.int32(MIN_I32))
    return lax.bitcast_convert_type(jnp.bitwise_xor(k16_i32, m), jnp.float32)


def _digit(k16_i32, shift):
    ku = lax.bitcast_convert_type(k16_i32, jnp.uint32)
    return (jnp.right_shift(ku, shift) & jnp.uint32(255)).astype(jnp.int32)


def _sort_body(x3_hbm, wl_hbm, k1_hbm, sv_hbm, ka, kb, hist, fbuf, wlv, k1v):
    wid = _wid()
    lanes = jnp.arange(L, dtype=jnp.int32)
    ones = jnp.ones((L,), jnp.int32)
    segbase = lanes * NV

    pltpu.sync_copy(wl_hbm.at[wid], wlv)
    pltpu.sync_copy(k1_hbm, k1v)
    k1 = jnp.max(k1v[...])

    def slot(t, _):
        pos = wid + t * NW

        def do_sort():
            _sort_one_row(x3_hbm, sv_hbm, wlv, t, pos, ka, kb, hist, fbuf,
                          lanes, ones, segbase)
        lax.cond(pos < k1, do_sort, lambda: None)
        return 0

    lax.fori_loop(0, RPW, slot, 0)


def _sort_one_row(x3_hbm, sv_hbm, wlv, t, pos, ka, kb, hist, fbuf,
                  lanes, ones, segbase):
        def load_chunk(ci, _):
            pltpu.sync_copy(x3_hbm.at[wlv.at[t, pl.ds(ci, 1)]], fbuf)

            @plsc.parallel_loop(0, CH // L, unroll=4)
            def _conv(j):
                ka[pl.ds(ci * CH + j * L, L)] = _to_key(fbuf[0, pl.ds(j * L, L)])
            return 0
        lax.fori_loop(0, NCHUNK, load_chunk, 0)

        for p in range(4):
            src = ka if p % 2 == 0 else kb
            dst = kb if p % 2 == 0 else ka
            shift = jnp.uint32(8 * p)

            def zero(i, _):
                hist[pl.ds(i * L, L)] = jnp.zeros((L,), jnp.int32)
                return 0
            lax.fori_loop(0, NBINS * L // L, zero, 0)

            def count(i, _):
                i4 = i * 4
                ks = [plsc.load_gather(src, [segbase + (i4 + u)])
                      for u in range(4)]
                idxs = [jnp.left_shift(_digit(k, shift), 4) + lanes
                        for k in ks]
                for idx in idxs:
                    plsc.addupdate_scatter(hist, [idx], ones)
                return 0
            lax.fori_loop(0, NV // 4, count, 0)

            def scan(i, carry):
                v = hist[pl.ds(i * L, L)]
                inc = plsc.cumsum(v)
                hist[pl.ds(i * L, L)] = inc - v + carry
                return carry + jnp.sum(v)
            lax.fori_loop(0, NBINS, scan, jnp.int32(0))

            def permute(i, _):
                i4 = i * 4
                for u in range(4):
                    k = plsc.load_gather(src, [segbase + (i4 + u)])
                    idx = jnp.left_shift(_digit(k, shift), 4) + lanes
                    pos = plsc.load_gather(hist, [idx])
                    plsc.store_scatter(dst, [pos], k)
                    plsc.addupdate_scatter(hist, [idx], ones)
                return 0
            lax.fori_loop(0, NV // 4, permute, 0)

        def store_chunk(ci, _):
            @plsc.parallel_loop(0, CH // L, unroll=4)
            def _conv(j):
                fbuf[0, pl.ds(j * L, L)] = _from_key(
                    ka[pl.ds(ci * CH + j * L, L)])
            pltpu.sync_copy(fbuf, sv_hbm.at[pl.ds(pos, 1), pl.ds(ci * CH, CH)])
            return 0
        lax.fori_loop(0, NCHUNK, store_chunk, 0)


def _match_body(x3_hbm, sv_hbm, alphaS_hbm, svbS_hbm, svpS_hbm, wlx_hbm,
                k2_hbm, mout_hbm, svb, svp, bidx, pidx, av, wlv, k2v, xc, oc,
                sem):
    wid = _wid()
    zeros16 = jnp.zeros((L,), jnp.int32)
    pltpu.sync_copy(wlx_hbm.at[wid], wlv)
    pltpu.sync_copy(k2_hbm, k2v)
    k2 = jnp.max(k2v[...])

    def slot(t, _):
        pos = wid + t * NW

        def match():
            pltpu.sync_copy(alphaS_hbm.at[pos], av)
            pltpu.sync_copy(svbS_hbm.at[pos], bidx)
            pltpu.sync_copy(svpS_hbm.at[pos], pidx)
            pltpu.async_copy(sv_hbm.at[bidx.at[pl.ds(0, 1)]], svb, sem).wait()
            pltpu.async_copy(sv_hbm.at[pidx.at[pl.ds(0, 1)]], svp, sem).wait()
            alpha = av[...]

            def chunk(ci, _):
                pltpu.sync_copy(x3_hbm.at[wlv.at[t, pl.ds(ci, 1)]], xc)

                @plsc.parallel_loop(0, CH // L, unroll=4)
                def _elem(j):
                    xv = xc[0, pl.ds(j * L, L)]
                    rk = jnp.zeros((L,), jnp.int32)
                    for bit in (32768, 16384, 8192, 4096, 2048, 1024, 512,
                                256, 128, 64, 32, 16, 8, 4, 2, 1):
                        nr = rk + bit
                        im = jnp.minimum(nr, jnp.int32(N)) - 1
                        v = plsc.load_gather(svb, [zeros16, im])
                        rk = jnp.where(v < xv, nr, rk)
                    m = plsc.load_gather(svp, [zeros16, rk])
                    oc[pl.ds(j * L, L)] = xv + alpha * (m - xv)
                pltpu.sync_copy(oc, mout_hbm.at[pos, pl.ds(ci * CH, CH)])
                return 0
            lax.fori_loop(0, NCHUNK, chunk, 0)

        lax.cond(pos < k2, match, lambda: None)
        return 0

    lax.fori_loop(0, RPW, slot, 0)


def _distribute_body(x_hbm, m3_hbm, alpha_hbm, posm_hbm, out_hbm,
                     av, pm, xc1, xc, sem):
    wid = _wid()

    def row(t, _):
        r = wid * RPW + t
        pltpu.sync_copy(alpha_hbm.at[r], av)
        a16 = av[...]
        active = jnp.sum(jnp.where(a16 != 0.0, jnp.int32(1), jnp.int32(0))) > 0

        def dist():
            pltpu.sync_copy(posm_hbm.at[r], pm)
            pltpu.async_copy(m3_hbm.at[pm.at[0]], xc1, sem).wait()
            pltpu.sync_copy(xc1, out_hbm.at[pl.ds(r, 1)])

        def copy():
            pltpu.sync_copy(x_hbm.at[r], xc)
            pltpu.sync_copy(xc, out_hbm.at[r])

        lax.cond(active, dist, copy)
        return 0

    lax.fori_loop(0, RPW, row, 0)


def kernel(CA, x):
    xr = x.reshape(R, N)

    perm = jnp.arange(B - 1, -1, -1)
    pk1, pk2 = jax.random.split(jax.random.key(42))
    perm_b = perm[: B // 2][jax.random.permutation(pk1, B // 2)]
    perm_a = perm[B // 2:][jax.random.permutation(pk2, B // 2)]
    perm = jnp.concatenate([perm_b, perm_a], axis=0)

    alpha = jax.nn.sigmoid(-999999.0 * (CA + CA[perm] - 0.6))
    alpha16 = jnp.broadcast_to(alpha.reshape(R, 1), (R, L)).astype(jnp.float32)
    srcrow = (perm[:, None] * C + jnp.arange(C)[None, :]).reshape(R)

    need = alpha.reshape(B, C) != 0.0
    pinv = jnp.argsort(perm)
    sortneed = (need | need[pinv]).reshape(R)

    orderA = jnp.argsort(1 - sortneed.astype(jnp.int32), stable=True)
    k1 = jnp.sum(sortneed.astype(jnp.int32)).astype(jnp.int32)
    k1v = jnp.full((L,), k1, jnp.int32)
    posA = jnp.argsort(orderA).astype(jnp.int32)
    slots = (jnp.arange(NW)[:, None] + jnp.arange(RPW)[None, :] * NW)
    wlA8 = (orderA[slots][..., None] * 8
            + jnp.arange(8)[None, None, :]).astype(jnp.int32)
    posb8 = jnp.broadcast_to(posA.reshape(R, 1), (R, 8)).astype(jnp.int32)
    posp8 = jnp.broadcast_to(posA[srcrow].reshape(R, 1), (R, 8)).astype(jnp.int32)

    needR = need.reshape(R)
    orderB = jnp.argsort(1 - needR.astype(jnp.int32), stable=True)
    k2 = jnp.sum(needR.astype(jnp.int32)).astype(jnp.int32)
    k2v = jnp.full((L,), k2, jnp.int32)
    posB = jnp.argsort(orderB).astype(jnp.int32)
    wlBx8 = (orderB[slots][..., None] * NCHUNK
             + jnp.arange(NCHUNK)[None, None, :]).astype(jnp.int32)
    alphaS = alpha16[orderB]
    svbS8 = posb8[orderB]
    svpS8 = posp8[orderB]
    posm8 = jnp.broadcast_to(
        posB.reshape(R, 1, 1), (R, NCHUNK, 1)).astype(jnp.int32)

    mesh = plsc.VectorSubcoreMesh(core_axis_name="c", subcore_axis_name="s")

    sv = pl.kernel(
        _sort_body,
        out_type=jax.ShapeDtypeStruct((R, N), jnp.float32),
        mesh=mesh,
        compiler_params=pltpu.CompilerParams(needs_layout_passes=False),
        scratch_types=[
            pltpu.VMEM((N,), jnp.int32),
            pltpu.VMEM((N,), jnp.int32),
            pltpu.VMEM((NBINS * L,), jnp.int32),
            pltpu.VMEM((1, CH), jnp.float32),
            pltpu.VMEM((RPW, 8), jnp.int32),
            pltpu.VMEM((L,), jnp.int32),
        ],
    )(xr.reshape(R * NCHUNK, CH), wlA8, k1v)

    mout = pl.kernel(
        _match_body,
        out_type=jax.ShapeDtypeStruct((R, N), jnp.float32),
        mesh=mesh,
        compiler_params=pltpu.CompilerParams(needs_layout_passes=False),
        scratch_types=[
            pltpu.VMEM((1, N), jnp.float32),
            pltpu.VMEM((1, N), jnp.float32),
            pltpu.VMEM((8,), jnp.int32),
            pltpu.VMEM((8,), jnp.int32),
            pltpu.VMEM((L,), jnp.float32),
            pltpu.VMEM((RPW, 8), jnp.int32),
            pltpu.VMEM((L,), jnp.int32),
            pltpu.VMEM((1, CH), jnp.float32),
            pltpu.VMEM((CH,), jnp.float32),
            pltpu.SemaphoreType.DMA,
        ],
    )(xr.reshape(R * NCHUNK, CH), sv, alphaS, svbS8, svpS8, wlBx8, k2v)

    out = pl.kernel(
        _distribute_body,
        out_type=jax.ShapeDtypeStruct((R, N), jnp.float32),
        mesh=mesh,
        compiler_params=pltpu.CompilerParams(needs_layout_passes=False),
        scratch_types=[
            pltpu.VMEM((L,), jnp.float32),
            pltpu.VMEM((NCHUNK, 1), jnp.int32),
            pltpu.VMEM((1, N), jnp.float32),
            pltpu.VMEM((N,), jnp.float32),
            pltpu.SemaphoreType.DMA,
        ],
    )(xr, mout, alpha16, posm8)

    return out.reshape(B, C, W, H)
